# baseline (speedup 1.0000x reference)
import jax
import jax.numpy as jnp
from jax import lax
from jax.experimental import pallas as pl
from jax.experimental.pallas import tpu as pltpu
from jax.experimental.pallas import tpu_sc as plsc

_B, _C, _H, _W = 32, 2, 512, 512
_K = 500
_KPAD = 512
_NIDX = _C * _KPAD
_LANES = 16
_BLK = 8
_MOFF = _KPAD
_TOFF = 2 * _KPAD
_BLOB = 4 * _KPAD


def _tec_body(feat_ref, blob_ref, out_ref,
              blob_v, live_v, vals8_v, part_v, sem, semb):
    c = lax.axis_index("c")
    s = lax.axis_index("s")
    b = s * 2 + c

    blob_cp = pltpu.async_copy(blob_ref.at[b], blob_v, semb)

    row0_base = b * _C * _H
    lane_ids = lax.iota(jnp.int32, _LANES)

    dummy = jnp.full((_LANES,), (_KPAD - 1) << 18, jnp.int32)
    for i in range(_KPAD // _LANES + 1):
        live_v[pl.ds(i * _LANES, _LANES)] = dummy
    blob_cp.wait()

    def compact(i, base):
        iv = blob_v[pl.ds(pl.multiple_of(i * _LANES, _LANES), _LANES)]
        mv = blob_v[pl.ds(pl.multiple_of(_MOFF + i * _LANES, _LANES),
                          _LANES)]
        mb = mv != 0
        m01 = jnp.where(mb, 1, 0)
        excl = plsc.cumsum(m01) - m01
        codes = iv | lax.shift_left(lane_ids + i * _LANES, 18)
        plsc.store_scatter(live_v, [excl + base], codes, mask=mb)
        return base + plsc.all_reduce_population_count(mb)[0]

    live = lax.fori_loop(0, _KPAD // _LANES, compact, 0)
    nchunk = lax.shift_right_logical(live + _LANES - 1, 4)

    @plsc.parallel_loop(0, nchunk, unroll=2)
    def issue(i):
        cv = live_v[pl.ds(pl.multiple_of(i * _LANES, _LANES), _LANES)]
        for j in range(_LANES):
            code = cv[j]
            p = code & (1 << 18) - 1
            k = lax.shift_right_logical(code, 18)
            h = lax.shift_right_logical(p, 9)
            w8 = pl.multiple_of(p & (_W - 1) & ~(_BLK - 1), _BLK)
            dst0 = pl.multiple_of(k * _BLK, _BLK)
            dst1 = pl.multiple_of((k + _KPAD) * _BLK, _BLK)
            r0 = row0_base + h
            pltpu.async_copy(feat_ref.at[r0, pl.ds(w8, _BLK)],
                             vals8_v.at[pl.ds(dst0, _BLK)], sem)
            pltpu.async_copy(feat_ref.at[r0 + _H, pl.ds(w8, _BLK)],
                             vals8_v.at[pl.ds(dst1, _BLK)], sem)

    def drain16(j, carry):
        pltpu.make_async_copy(feat_ref.at[0, pl.ds(0, 16 * _BLK)],
                              vals8_v.at[pl.ds(0, 16 * _BLK)], sem).wait()
        return carry

    lax.fori_loop(0, 2 * nchunk, drain16, 0)

    acc = jnp.zeros((_LANES,), jnp.float32)
    macc = jnp.zeros((_LANES,), jnp.float32)
    for i in range(_KPAD // _LANES):
        l = blob_v[pl.ds(i * _LANES, _LANES)] & (_BLK - 1)
        k_vec = lane_ids + i * _LANES
        p0 = plsc.load_gather(vals8_v, [k_vec * _BLK + l])
        p1 = plsc.load_gather(vals8_v, [(k_vec + _KPAD) * _BLK + l])
        mi = blob_v[pl.ds(_MOFF + i * _LANES, _LANES)]
        t0 = plsc.bitcast(
            plsc.load_gather(blob_v, [_TOFF + 2 * k_vec]), jnp.float32)
        t1 = plsc.bitcast(
            plsc.load_gather(blob_v, [_TOFF + 2 * k_vec + 1]), jnp.float32)
        d0 = jnp.abs(p0 - t0)
        d1 = jnp.abs(p1 - t1)
        acc = acc + jnp.where(mi != 0, d0 + d1, 0.0)
        macc = macc + mi.astype(jnp.float32)

    part_v[pl.ds(0, _LANES)] = acc
    part_v[pl.ds(_LANES, _LANES)] = macc
    pltpu.sync_copy(part_v, out_ref.at[b])


@jax.jit
def kernel(output, mask, ind, target):
    feat = output.reshape(_B * _C * _H, _W)
    tgt_bits = lax.bitcast_convert_type(target, jnp.int32).reshape(_B, -1)
    pad = ((0, 0), (0, _KPAD - _K))
    blob = jnp.concatenate([
        jnp.pad(ind, pad),
        jnp.pad(mask.astype(jnp.int32), pad),
        jnp.pad(tgt_bits, ((0, 0), (0, 2 * _KPAD - _C * _K))),
    ], axis=1)

    mesh = plsc.VectorSubcoreMesh(core_axis_name="c", subcore_axis_name="s")
    f = pl.kernel(
        _tec_body,
        mesh=mesh,
        compiler_params=pltpu.CompilerParams(needs_layout_passes=False),
        out_type=jax.ShapeDtypeStruct((_B, 2 * _LANES), jnp.float32),
        scratch_types=[
            pltpu.VMEM((_BLOB,), jnp.int32),
            pltpu.VMEM((_KPAD + _LANES,), jnp.int32),
            pltpu.VMEM((_NIDX * _BLK,), jnp.float32),
            pltpu.VMEM((2 * _LANES,), jnp.float32),
            pltpu.SemaphoreType.DMA,
            pltpu.SemaphoreType.DMA,
        ],
    )
    parts = f(feat, blob)
    loss = jnp.sum(parts[:, :_LANES]) / (
        _C * jnp.sum(parts[:, _LANES:]) + 1e-4)
    return loss


# speedup vs baseline: 1.1108x; 1.1108x over previous
import jax
import jax.numpy as jnp
from jax import lax
from jax.experimental import pallas as pl
from jax.experimental.pallas import tpu as pltpu
from jax.experimental.pallas import tpu_sc as plsc

_B, _C, _H, _W = 32, 2, 512, 512
_K = 500
_KPAD = 512
_NIDX = _C * _KPAD
_LANES = 16
_BLK = 8
_MOFF = _KPAD
_TOFF = 2 * _KPAD
_BLOB = 4 * _KPAD


def _tec_body(feat_ref, blob_ref, out_ref,
              blob_v, live_v, vals8_v, part_v, sem, semb):
    c = lax.axis_index("c")
    s = lax.axis_index("s")
    b = s * 2 + c

    blob_cp = pltpu.async_copy(blob_ref.at[b], blob_v, semb)

    row0_base = b * _C * _H
    lane_ids = lax.iota(jnp.int32, _LANES)

    dummy = jnp.full((_LANES,), (_KPAD - 1) << 18, jnp.int32)
    for i in range(_KPAD // _LANES + 1):
        live_v[pl.ds(i * _LANES, _LANES)] = dummy
    blob_cp.wait()

    def compact(i, base):
        iv = blob_v[pl.ds(pl.multiple_of(i * _LANES, _LANES), _LANES)]
        mv = blob_v[pl.ds(pl.multiple_of(_MOFF + i * _LANES, _LANES),
                          _LANES)]
        mb = mv != 0
        m01 = jnp.where(mb, 1, 0)
        excl = plsc.cumsum(m01) - m01
        codes = iv | lax.shift_left(lane_ids + i * _LANES, 18)
        plsc.store_scatter(live_v, [excl + base], codes, mask=mb)
        return base + plsc.all_reduce_population_count(mb)[0]

    live = lax.fori_loop(0, _KPAD // _LANES, compact, 0)
    nchunk = lax.shift_right_logical(live + _LANES - 1, 4)

    def issue(i, carry):
        cv = live_v[pl.ds(pl.multiple_of(i * _LANES, _LANES), _LANES)]
        for j in range(_LANES):
            code = cv[j]
            p = code & (1 << 18) - 1
            k = lax.shift_right_logical(code, 18)
            h = lax.shift_right_logical(p, 9)
            w8 = pl.multiple_of(p & (_W - 1) & ~(_BLK - 1), _BLK)
            dst0 = pl.multiple_of(k * _BLK, _BLK)
            dst1 = pl.multiple_of((k + _KPAD) * _BLK, _BLK)
            r0 = row0_base + h
            pltpu.async_copy(feat_ref.at[r0, pl.ds(w8, _BLK)],
                             vals8_v.at[pl.ds(dst0, _BLK)], sem)
            pltpu.async_copy(feat_ref.at[r0 + _H, pl.ds(w8, _BLK)],
                             vals8_v.at[pl.ds(dst1, _BLK)], sem)
        return carry

    lax.fori_loop(0, nchunk, issue, 0)

    def drain16(j, carry):
        pltpu.make_async_copy(feat_ref.at[0, pl.ds(0, 16 * _BLK)],
                              vals8_v.at[pl.ds(0, 16 * _BLK)], sem).wait()
        return carry

    lax.fori_loop(0, 2 * nchunk, drain16, 0)

    acc = jnp.zeros((_LANES,), jnp.float32)
    macc = jnp.zeros((_LANES,), jnp.float32)
    for i in range(_KPAD // _LANES):
        l = blob_v[pl.ds(i * _LANES, _LANES)] & (_BLK - 1)
        k_vec = lane_ids + i * _LANES
        p0 = plsc.load_gather(vals8_v, [k_vec * _BLK + l])
        p1 = plsc.load_gather(vals8_v, [(k_vec + _KPAD) * _BLK + l])
        mi = blob_v[pl.ds(_MOFF + i * _LANES, _LANES)]
        t0 = plsc.bitcast(
            plsc.load_gather(blob_v, [_TOFF + 2 * k_vec]), jnp.float32)
        t1 = plsc.bitcast(
            plsc.load_gather(blob_v, [_TOFF + 2 * k_vec + 1]), jnp.float32)
        d0 = jnp.abs(p0 - t0)
        d1 = jnp.abs(p1 - t1)
        acc = acc + jnp.where(mi != 0, d0 + d1, 0.0)
        macc = macc + mi.astype(jnp.float32)

    part_v[pl.ds(0, _LANES)] = acc
    part_v[pl.ds(_LANES, _LANES)] = macc
    pltpu.sync_copy(part_v, out_ref.at[b])


@jax.jit
def kernel(output, mask, ind, target):
    feat = output.reshape(_B * _C * _H, _W)
    tgt_bits = lax.bitcast_convert_type(target, jnp.int32).reshape(_B, -1)
    pad = ((0, 0), (0, _KPAD - _K))
    blob = jnp.concatenate([
        jnp.pad(ind, pad),
        jnp.pad(mask.astype(jnp.int32), pad),
        jnp.pad(tgt_bits, ((0, 0), (0, 2 * _KPAD - _C * _K))),
    ], axis=1)

    mesh = plsc.VectorSubcoreMesh(core_axis_name="c", subcore_axis_name="s")
    f = pl.kernel(
        _tec_body,
        mesh=mesh,
        compiler_params=pltpu.CompilerParams(needs_layout_passes=False),
        out_type=jax.ShapeDtypeStruct((_B, 2 * _LANES), jnp.float32),
        scratch_types=[
            pltpu.VMEM((_BLOB,), jnp.int32),
            pltpu.VMEM((_KPAD + _LANES,), jnp.int32),
            pltpu.VMEM((_NIDX * _BLK,), jnp.float32),
            pltpu.VMEM((2 * _LANES,), jnp.float32),
            pltpu.SemaphoreType.DMA,
            pltpu.SemaphoreType.DMA,
        ],
    )
    parts = f(feat, blob)
    loss = jnp.sum(parts[:, :_LANES]) / (
        _C * jnp.sum(parts[:, _LANES:]) + 1e-4)
    return loss
